# baseline (speedup 1.0000x reference)
import jax
import jax.numpy as jnp
from jax.experimental import pallas as pl
from jax.experimental.pallas import tpu as pltpu

_MU = 3.5
_N_ITEMS = 2048
_N_USERS = 2048
_BLK = 256
_NBLK = _N_USERS // _BLK
_KTOP = 100


def _nbm_kernel(sref, r_blk, r_urow, w_row, o_row, ib_row, ub_row, out,
                acc_ss, acc_dot):
    g = pl.program_id(0)
    u = sref[0]
    i = sref[1]
    kval = sref[2]
    lane = jax.lax.broadcasted_iota(jnp.int32, (1, _N_ITEMS), 1)

    @pl.when(g == 0)
    def _init():
        acc_ss[...] = jnp.zeros_like(acc_ss)
        acc_dot[...] = jnp.zeros_like(acc_dot)

    blk = r_blk[...]
    oh = (lane == i).astype(jnp.float32)
    c = jax.lax.dot_general(blk, oh, (((1,), (1,)), ((), ())),
                            precision=jax.lax.Precision.HIGHEST,
                            preferred_element_type=jnp.float32)
    acc_ss[...] += jnp.sum(blk * blk, axis=0, keepdims=True)
    acc_dot[...] += jax.lax.dot_general(c, blk, (((0,), (0,)), ((), ())),
                                        precision=jax.lax.Precision.HIGHEST,
                                        preferred_element_type=jnp.float32)

    @pl.when(g == _NBLK - 1)
    def _finalize():
        ss = acc_ss[...]
        dt = acc_dot[...]
        norm = jnp.sqrt(ss)
        safe = jnp.where(norm == 0.0, 1.0, norm)
        n_i = jnp.sum(jnp.where(lane == i, norm, 0.0))
        safe_i = jnp.where(n_i == 0.0, 1.0, n_i)
        s_row = dt / (safe * safe_i)

        neg = jnp.float32(-3.4e38)

        def body(t, carry):
            cur, selm = carry
            m = jnp.max(cur)
            j = jnp.min(jnp.where(cur == m, lane, _N_ITEMS))
            pick = lane == j
            cur = jnp.where(pick, neg, cur)
            take = pick & (t < kval)
            selm = jnp.where(take, 1.0, selm)
            return cur, selm

        sel0 = jnp.zeros((1, _N_ITEMS), dtype=jnp.float32)
        _, topk = jax.lax.fori_loop(0, _KTOP, body, (s_row, sel0))

        ru = r_urow[0]
        selm = (topk > 0.5) & (ru != 0.0)
        num_k = jnp.sum(selm.astype(jnp.float32))
        ib = ib_row[...]
        bu = jnp.sum(jnp.where(lane == u, ub_row[...], 0.0))
        bi = jnp.sum(jnp.where(lane == i, ib, 0.0))
        buj = _MU + bu + ib
        ruj = jnp.floor(ru)
        s1 = jnp.sum(jnp.where(selm, (ruj - buj) * w_row[0], 0.0))
        s2 = jnp.sum(jnp.where(selm, o_row[0], 0.0))
        nrm = jax.lax.rsqrt(num_k)
        rui = _MU + bu + bi + nrm * s1 + nrm * s2
        out[...] = jnp.reshape(rui, (1, 1))


def kernel(R, user, item, item_weights, implicit_offset, user_biases,
           item_biases, k):
    u32 = user.astype(jnp.int32)[0]
    i32 = item.astype(jnp.int32)[0]
    k32 = jnp.asarray(k, jnp.int32)
    sref = jnp.stack([u32, i32, k32])
    ib = item_biases.reshape(1, _N_ITEMS)
    ub = user_biases.reshape(1, _N_USERS)
    R3 = R.reshape(_N_USERS, 1, _N_ITEMS)
    W3 = item_weights.reshape(_N_ITEMS, 1, _N_ITEMS)
    O3 = implicit_offset.reshape(_N_ITEMS, 1, _N_ITEMS)

    grid_spec = pltpu.PrefetchScalarGridSpec(
        num_scalar_prefetch=1,
        grid=(_NBLK,),
        in_specs=[
            pl.BlockSpec((_BLK, _N_ITEMS), lambda g, s: (g, 0)),
            pl.BlockSpec((1, 1, _N_ITEMS), lambda g, s: (s[0], 0, 0)),
            pl.BlockSpec((1, 1, _N_ITEMS), lambda g, s: (s[1], 0, 0)),
            pl.BlockSpec((1, 1, _N_ITEMS), lambda g, s: (s[1], 0, 0)),
            pl.BlockSpec((1, _N_ITEMS), lambda g, s: (0, 0)),
            pl.BlockSpec((1, _N_USERS), lambda g, s: (0, 0)),
        ],
        out_specs=pl.BlockSpec((1, 1), lambda g, s: (0, 0)),
        scratch_shapes=[pltpu.VMEM((1, _N_ITEMS), jnp.float32)] * 2,
    )
    out = pl.pallas_call(
        _nbm_kernel,
        grid_spec=grid_spec,
        out_shape=jax.ShapeDtypeStruct((1, 1), jnp.float32),
    )(sref, R, R3, W3, O3, ib, ub)
    return out[0, 0]


# speedup vs baseline: 1.3417x; 1.3417x over previous
import jax
import jax.numpy as jnp
from jax.experimental import pallas as pl
from jax.experimental.pallas import tpu as pltpu

_MU = 3.5
_N_ITEMS = 2048
_N_USERS = 2048
_BLK = 256
_NBLK = _N_USERS // _BLK
_KTOP = 100


def _nbm_kernel(sref, r_blk, r_urow, w_row, o_row, ib_row, ub_row, out,
                acc_ss, acc_dot):
    g = pl.program_id(0)
    u = sref[0]
    i = sref[1]
    kval = sref[2]
    lane = jax.lax.broadcasted_iota(jnp.int32, (1, _N_ITEMS), 1)

    @pl.when(g == 0)
    def _init():
        acc_ss[...] = jnp.zeros_like(acc_ss)
        acc_dot[...] = jnp.zeros_like(acc_dot)

    blk = r_blk[...]
    oh = (lane == i).astype(jnp.float32)
    c = jax.lax.dot_general(blk, oh, (((1,), (1,)), ((), ())),
                            precision=jax.lax.Precision.HIGHEST,
                            preferred_element_type=jnp.float32)
    ones = jnp.ones((1, _BLK), jnp.float32)
    sq = blk * blk
    acc_ss[...] += jax.lax.dot_general(ones, sq, (((1,), (0,)), ((), ())),
                                       precision=jax.lax.Precision.HIGHEST,
                                       preferred_element_type=jnp.float32)
    acc_dot[...] += jax.lax.dot_general(c, blk, (((0,), (0,)), ((), ())),
                                        precision=jax.lax.Precision.HIGHEST,
                                        preferred_element_type=jnp.float32)

    @pl.when(g == _NBLK - 1)
    def _finalize():
        ss = acc_ss[...]
        dt = acc_dot[...]
        norm = jnp.sqrt(ss)
        safe = jnp.where(norm == 0.0, 1.0, norm)
        n_i = jnp.sum(jnp.where(lane == i, norm, 0.0))
        safe_i = jnp.where(n_i == 0.0, 1.0, n_i)
        s_row = dt / (safe * safe_i)

        r2, c2 = 16, _N_ITEMS // 16
        s2d = jnp.reshape(s_row, (r2, c2))
        fidx = (jax.lax.broadcasted_iota(jnp.int32, (r2, c2), 0) * c2
                + jax.lax.broadcasted_iota(jnp.int32, (r2, c2), 1))

        bits = jax.lax.bitcast_convert_type(s2d, jnp.int32)
        skey = bits ^ (jnp.right_shift(bits, 31) & jnp.int32(0x7FFFFFFF))
        thr = jnp.int32(-(1 << 31))
        for b in range(31, -1, -1):
            step = (1 << b) - (1 << 32) if b == 31 else (1 << b)
            cand = thr + jnp.int32(step)
            cnt = jnp.sum((skey >= cand).astype(jnp.int32))
            thr = jnp.where(cnt >= kval, cand, thr)
        gt = skey > thr
        needed = kval - jnp.sum(gt.astype(jnp.int32))
        eq = skey == thr
        jcut = jnp.int32(0)
        for b in range(11, -1, -1):
            candj = jcut + jnp.int32(1 << b)
            cntj = jnp.sum((eq & (fidx < candj)).astype(jnp.int32))
            jcut = jnp.where(cntj <= needed, candj, jcut)
        topk = gt | (eq & (fidx < jcut))

        ru = jnp.reshape(r_urow[0], (r2, c2))
        w2 = jnp.reshape(w_row[0], (r2, c2))
        o2 = jnp.reshape(o_row[0], (r2, c2))
        ib2 = jnp.reshape(ib_row[...], (r2, c2))
        ub2 = jnp.reshape(ub_row[...], (r2, c2))
        selm = topk & (ru != 0.0)
        num_k = jnp.sum(selm.astype(jnp.float32))
        bu = jnp.sum(jnp.where(fidx == u, ub2, 0.0))
        bi = jnp.sum(jnp.where(fidx == i, ib2, 0.0))
        buj = _MU + bu + ib2
        ruj = jnp.floor(ru)
        s1 = jnp.sum(jnp.where(selm, (ruj - buj) * w2, 0.0))
        s2 = jnp.sum(jnp.where(selm, o2, 0.0))
        nrm = jax.lax.rsqrt(num_k)
        rui = _MU + bu + bi + nrm * s1 + nrm * s2
        out[...] = jnp.reshape(rui, (1, 1))


def kernel(R, user, item, item_weights, implicit_offset, user_biases,
           item_biases, k):
    u32 = user.astype(jnp.int32)[0]
    i32 = item.astype(jnp.int32)[0]
    k32 = jnp.asarray(k, jnp.int32)
    sref = jnp.stack([u32, i32, k32])
    ib = item_biases.reshape(1, _N_ITEMS)
    ub = user_biases.reshape(1, _N_USERS)
    R3 = R.reshape(_N_USERS, 1, _N_ITEMS)
    W3 = item_weights.reshape(_N_ITEMS, 1, _N_ITEMS)
    O3 = implicit_offset.reshape(_N_ITEMS, 1, _N_ITEMS)

    grid_spec = pltpu.PrefetchScalarGridSpec(
        num_scalar_prefetch=1,
        grid=(_NBLK,),
        in_specs=[
            pl.BlockSpec((_BLK, _N_ITEMS), lambda g, s: (g, 0)),
            pl.BlockSpec((1, 1, _N_ITEMS), lambda g, s: (s[0], 0, 0)),
            pl.BlockSpec((1, 1, _N_ITEMS), lambda g, s: (s[1], 0, 0)),
            pl.BlockSpec((1, 1, _N_ITEMS), lambda g, s: (s[1], 0, 0)),
            pl.BlockSpec((1, _N_ITEMS), lambda g, s: (0, 0)),
            pl.BlockSpec((1, _N_USERS), lambda g, s: (0, 0)),
        ],
        out_specs=pl.BlockSpec((1, 1), lambda g, s: (0, 0)),
        scratch_shapes=[pltpu.VMEM((1, _N_ITEMS), jnp.float32)] * 2,
    )
    out = pl.pallas_call(
        _nbm_kernel,
        grid_spec=grid_spec,
        out_shape=jax.ShapeDtypeStruct((1, 1), jnp.float32),
    )(sref, R, R3, W3, O3, ib, ub)
    return out[0, 0]
